# baseline (speedup 1.0000x reference)
import functools

import jax
import jax.numpy as jnp
from jax import lax
from jax.experimental import pallas as pl
from jax.experimental.pallas import tpu as pltpu
from jax.experimental.pallas import tpu_sc as plsc

_ROWS = 1152
_LANES = 128
_N = _ROWS * _LANES
_NI = 27
_CHUNK = 32
_NCH = _ROWS // _CHUNK
_NW = 32
_P = _N // _NW
_BINS = _NI * 16


def _sc_stats_body(b, tgt_ref, pred_ref, out_ref,
                   t0v, t1v, t2v, xv, yv, zv, bins0, bins1, sem):
    wid = lax.axis_index("s") * 2 + lax.axis_index("c")
    base = wid * _P
    lane = lax.broadcasted_iota(jnp.int32, (16,), 0)
    ones16 = jnp.ones((16,), jnp.float32)
    z16 = jnp.zeros((16,), jnp.float32)
    cps = [
        pltpu.async_copy(tgt_ref.at[pl.ds(b * 3 * _N + base, _P)], t0v, sem),
        pltpu.async_copy(tgt_ref.at[pl.ds((b * 3 + 1) * _N + base, _P)], t1v, sem),
        pltpu.async_copy(tgt_ref.at[pl.ds((b * 3 + 2) * _N + base, _P)], t2v, sem),
        pltpu.async_copy(pred_ref.at[pl.ds(b * 3 * _N + base, _P)], xv, sem),
        pltpu.async_copy(pred_ref.at[pl.ds((b * 3 + 1) * _N + base, _P)], yv, sem),
        pltpu.async_copy(pred_ref.at[pl.ds((b * 3 + 2) * _N + base, _P)], zv, sem),
    ]
    for k in range(4 * _BINS // 16):
        sl = pl.ds(k * 16, 16)
        bins0[sl] = z16
        bins1[sl] = z16
    for c in cps:
        c.wait()

    unroll = 8

    def body(i, carry):
        for u in range(unroll):
            sl = pl.ds((i * unroll + u) * 16, 16)
            sid = t0v[sl] * 9 + t1v[sl] * 3 + t2v[sl]
            idx = sid * 16 + lane
            bset = bins0 if u % 2 == 0 else bins1
            plsc.addupdate_scatter(bset, [idx], ones16)
            plsc.addupdate_scatter(bset, [idx + _BINS], xv[sl])
            plsc.addupdate_scatter(bset, [idx + 2 * _BINS], yv[sl])
            plsc.addupdate_scatter(bset, [idx + 3 * _BINS], zv[sl])
        return carry

    lax.fori_loop(0, _P // (16 * unroll), body, 0)
    for si, bset in enumerate((bins0, bins1)):
        for q in range(4):
            pltpu.sync_copy(
                bset.at[pl.ds(q * _BINS, _BINS)],
                out_ref.at[pl.ds(((q * 2 + si) * _NW + wid) * _BINS, _BINS)])


def _sc_stats(tgt_flat, pred_flat, b):
    mesh = plsc.VectorSubcoreMesh(core_axis_name="c", subcore_axis_name="s",
                                  num_cores=2, num_subcores=16)
    vm = pltpu.VMEM
    call = pl.kernel(
        functools.partial(_sc_stats_body, b),
        out_type=jax.ShapeDtypeStruct((4 * 2 * _NW * _BINS,), jnp.float32),
        mesh=mesh,
        compiler_params=pltpu.CompilerParams(needs_layout_passes=False),
        scratch_types=[
            vm((_P,), jnp.int32), vm((_P,), jnp.int32), vm((_P,), jnp.int32),
            vm((_P,), jnp.float32), vm((_P,), jnp.float32), vm((_P,), jnp.float32),
            vm((4 * _BINS,), jnp.float32), vm((4 * _BINS,), jnp.float32),
            pltpu.SemaphoreType.DMA,
        ],
    )
    return call(tgt_flat, pred_flat).reshape(4, 2 * _NW, _BINS)


def _loss_body(nobg_ref, pred_ref, tgt_ref, part_ref, out_ref,
               stats_ref, acc_ref, tot_ref, sid_ref, hub_ref):
    b = pl.program_id(0)
    i = pl.program_id(1)
    f32 = jnp.float32

    @pl.when(i == 0)
    def _combine():
        sid_ref[...] = tgt_ref[0, 0] * 9 + tgt_ref[0, 1] * 3 + tgt_ref[0, 2]

        @pl.when(b == 0)
        def _init_tot():
            tot_ref[0] = f32(0.0)

        ri = lax.broadcasted_iota(jnp.int32, (_BINS, 32), 0)
        ci = lax.broadcasted_iota(jnp.int32, (_BINS, 32), 1)
        fold = (lax.shift_right_logical(ri, 4) == ci).astype(f32)
        lanes32 = lax.broadcasted_iota(jnp.int32, (1, 32), 1)
        zv = jnp.zeros((1, 32), f32)
        for q in range(4):
            arr = part_ref[0, q]
            m1 = lax.dot_general(arr, fold, (((1,), (0,)), ((), ())),
                                 precision=lax.Precision.HIGHEST,
                                 preferred_element_type=f32)
            v = jnp.sum(m1, axis=0, keepdims=True)
            for j in range(_NI):
                stats_ref[j, q] = jnp.sum(jnp.where(lanes32 == j, v, zv))

    @pl.when(i == 1)
    def _gather_huber():
        mus = [(f32(0.0), f32(0.0), f32(0.0))]
        stats_ref[0, 4] = f32(0.0)
        stats_ref[0, 5] = f32(0.0)
        stats_ref[0, 6] = f32(0.0)
        for j in range(1, _NI):
            safe = jnp.maximum(stats_ref[j, 0], 1.0)
            mj = (stats_ref[j, 1] / safe,
                  stats_ref[j, 2] / safe,
                  stats_ref[j, 3] / safe)
            stats_ref[j, 4] = mj[0]
            stats_ref[j, 5] = mj[1]
            stats_ref[j, 6] = mj[2]
            mus.append(mj)
        for c in range(_NCH):
            sl = pl.ds(c * _CHUNK, _CHUNK)
            sid = sid_ref[sl]
            zc = jnp.zeros((_CHUNK, _LANES), f32)
            mx, my, mz = zc, zc, zc
            for j in range(1, _NI):
                m = sid == j
                mx = jnp.where(m, mus[j][0], mx)
                my = jnp.where(m, mus[j][1], my)
                mz = jnp.where(m, mus[j][2], mz)
            dx = pred_ref[0, 0, sl] - mx
            dy = pred_ref[0, 1, sl] - my
            dz = pred_ref[0, 2, sl] - mz
            adx = jnp.abs(dx)
            ady = jnp.abs(dy)
            adz = jnp.abs(dz)
            nx = jnp.minimum(adx, 1.0)
            ny = jnp.minimum(ady, 1.0)
            nz = jnp.minimum(adz, 1.0)
            hub = (nx * (2.0 * adx - nx) + ny * (2.0 * ady - ny)
                   + nz * (2.0 * adz - nz))
            hub_ref[sl] = 0.5 * hub

    @pl.when(i > 1)
    def _dense():
        j = i - 2
        cnt = stats_ref[j, 0]
        mex = stats_ref[j, 4]
        mey = stats_ref[j, 5]
        mez = stats_ref[j, 6]
        zc = jnp.zeros((_CHUNK, _LANES), f32)
        sa, ha, oa = zc, zc, zc
        for c in range(_NCH):
            sl = pl.ds(c * _CHUNK, _CHUNK)
            m = sid_ref[sl] == j
            dx = pred_ref[0, 0, sl] - mex
            dy = pred_ref[0, 1, sl] - mey
            dz = pred_ref[0, 2, sl] - mez
            dist = dx * dx + dy * dy + dz * dz
            fr = 300.0 / (1.0 + dist)
            sa = sa + fr
            ha = ha + jnp.where(m, hub_ref[sl], zc)
            oa = oa + jnp.where(m, fr, zc)
        Sj = jnp.sum(sa)
        Hj = jnp.sum(ha)
        OWNj = jnp.sum(oa)
        lanes = lax.broadcasted_iota(jnp.int32, (1, _LANES), 1)
        lm = lanes == j
        acc_ref[0:1] = jnp.where(lm, cnt, acc_ref[0:1])
        acc_ref[1:2] = jnp.where(lm, Hj, acc_ref[1:2])
        acc_ref[2:3] = jnp.where(lm, Sj, acc_ref[2:3])
        acc_ref[3:4] = jnp.where(lm, OWNj, acc_ref[3:4])
        acc_ref[4:5] = jnp.where(lm, mex, acc_ref[4:5])
        acc_ref[5:6] = jnp.where(lm, mey, acc_ref[5:6])
        acc_ref[6:7] = jnp.where(lm, mez, acc_ref[6:7])

        @pl.when(j == _NI - 1)
        def _assemble():
            lanes1 = lax.broadcasted_iota(jnp.int32, (1, _LANES), 1)
            inrange = lanes1 < _NI
            nobg_ok = nobg_ref[b] == 0
            cntv = acc_ref[0:1]
            Hv = acc_ref[1:2]
            Sv = acc_ref[2:3]
            OWNv = acc_ref[3:4]
            safev = jnp.maximum(cntv, 1.0)
            presentv = jnp.logical_and(cntv > 0.0, inrange)
            hmask = jnp.logical_and(presentv,
                                    jnp.logical_or(lanes1 > 0, nobg_ok))
            hterm = Hv / (safev * 3.0)
            ncv = f32(_N) - cntv
            sepv = ((Sv - OWNv) / jnp.maximum(ncv, 1.0)) * (10.0 / jnp.sqrt(safev))
            sepmask = jnp.logical_and(
                jnp.logical_and(presentv, ncv > 0.0), lanes1 > 0)
            zl = jnp.zeros_like(hterm)
            vv = jnp.where(hmask, jnp.ones_like(hterm), zl)
            loss = jnp.sum(jnp.where(hmask, hterm, zl)
                           + jnp.where(sepmask, sepv, zl))
            ct = jnp.sum(vv)

            onesv = jnp.ones((1, _LANES), f32)

            def _outer(v):
                return lax.dot_general(v, onesv, (((0,), (0,)), ((), ())),
                                       precision=lax.Precision.HIGHEST,
                                       preferred_element_type=f32)

            mxv = acc_ref[4:5]
            myv = acc_ref[5:6]
            mzv = acc_ref[6:7]
            ddx = _outer(mxv) - jnp.broadcast_to(mxv, (_LANES, _LANES))
            ddy = _outer(myv) - jnp.broadcast_to(myv, (_LANES, _LANES))
            ddz = _outer(mzv) - jnp.broadcast_to(mzv, (_LANES, _LANES))
            sq = ddx * ddx + ddy * ddy + ddz * ddz
            vcol = _outer(vv)
            vrow = jnp.broadcast_to(vv, (_LANES, _LANES))
            pv = vcol * vrow
            ri = lax.broadcasted_iota(jnp.int32, (_LANES, _LANES), 0)
            ci = lax.broadcasted_iota(jnp.int32, (_LANES, _LANES), 1)
            upper = jnp.logical_and(ri < ci, ci < _NI)
            zz = jnp.zeros_like(sq)
            pair_sum = jnp.sum(jnp.where(upper, (300.0 / (sq + 1.0)) * pv, zz))
            npair = jnp.sum(jnp.where(upper, pv, zz))
            pair_term = pair_sum / jnp.maximum(npair, 1.0)

            lossb = loss + jnp.where(ct > 1.0, pair_term, 0.0)
            tot_ref[0] += lossb / jnp.maximum(ct, 1.0)

            @pl.when(b == 1)
            def _finish():
                out_ref[...] = jnp.full((8, _LANES), tot_ref[0] * 0.5, f32)


def _make_call(interpret=False):
    return pl.pallas_call(
        _loss_body,
        grid=(2, _NI + 2),
        out_shape=jax.ShapeDtypeStruct((8, _LANES), jnp.float32),
        in_specs=[
            pl.BlockSpec(memory_space=pltpu.SMEM),
            pl.BlockSpec((1, 3, _ROWS, _LANES), lambda b, i: (b, 0, 0, 0)),
            pl.BlockSpec((1, 3, _ROWS, _LANES), lambda b, i: (b, 0, 0, 0)),
            pl.BlockSpec((1, 4, 2 * _NW, _BINS), lambda b, i: (b, 0, 0, 0)),
        ],
        out_specs=pl.BlockSpec((8, _LANES), lambda b, i: (0, 0)),
        scratch_shapes=[
            pltpu.SMEM((32, 8), jnp.float32),
            pltpu.VMEM((8, _LANES), jnp.float32),
            pltpu.SMEM((1,), jnp.float32),
            pltpu.VMEM((_ROWS, _LANES), jnp.int32),
            pltpu.VMEM((_ROWS, _LANES), jnp.float32),
        ],
        interpret=interpret,
    )


def kernel(prediction, target, no_bg):
    pred = prediction.astype(jnp.float32).reshape(2, 3, _ROWS, _LANES)
    tgt = target.astype(jnp.int32).reshape(2, 3, _ROWS, _LANES)
    tgt_flat = target.astype(jnp.int32).reshape(-1)
    pred_flat = prediction.astype(jnp.float32).reshape(-1)
    nobg = no_bg.astype(jnp.int32)
    part0 = _sc_stats(tgt_flat, pred_flat, 0)
    part1 = _sc_stats(tgt_flat, pred_flat, 1)
    part = jnp.stack([part0, part1])
    out = _make_call()(nobg, pred, tgt, part)
    return out[0, 0]


# speedup vs baseline: 1.0875x; 1.0875x over previous
import functools

import jax
import jax.numpy as jnp
from jax import lax
from jax.experimental import pallas as pl
from jax.experimental.pallas import tpu as pltpu
from jax.experimental.pallas import tpu_sc as plsc

_ROWS = 1152
_LANES = 128
_N = _ROWS * _LANES
_NI = 27
_CHUNK = 32
_NCH = _ROWS // _CHUNK
_NW = 32
_P = _N // _NW
_BINS = _NI * 16


def _sc_stats_body(b, tgt_ref, pred_ref, out_ref,
                   t0v, t1v, t2v, xv, yv, zv, bins0, bins1, sem):
    wid = lax.axis_index("s") * 2 + lax.axis_index("c")
    base = wid * _P
    lane = lax.broadcasted_iota(jnp.int32, (16,), 0)
    ones16 = jnp.ones((16,), jnp.float32)
    z16 = jnp.zeros((16,), jnp.float32)
    cps = [
        pltpu.async_copy(tgt_ref.at[pl.ds(b * 3 * _N + base, _P)], t0v, sem),
        pltpu.async_copy(tgt_ref.at[pl.ds((b * 3 + 1) * _N + base, _P)], t1v, sem),
        pltpu.async_copy(tgt_ref.at[pl.ds((b * 3 + 2) * _N + base, _P)], t2v, sem),
        pltpu.async_copy(pred_ref.at[pl.ds(b * 3 * _N + base, _P)], xv, sem),
        pltpu.async_copy(pred_ref.at[pl.ds((b * 3 + 1) * _N + base, _P)], yv, sem),
        pltpu.async_copy(pred_ref.at[pl.ds((b * 3 + 2) * _N + base, _P)], zv, sem),
    ]
    for k in range(4 * _BINS // 16):
        sl = pl.ds(k * 16, 16)
        bins0[sl] = z16
        bins1[sl] = z16
    for c in cps:
        c.wait()

    unroll = 8

    def body(i, carry):
        for u in range(unroll):
            sl = pl.ds((i * unroll + u) * 16, 16)
            sid = t0v[sl] * 9 + t1v[sl] * 3 + t2v[sl]
            idx = sid * 16 + lane
            bset = bins0 if u % 2 == 0 else bins1
            plsc.addupdate_scatter(bset, [idx], ones16)
            plsc.addupdate_scatter(bset, [idx + _BINS], xv[sl])
            plsc.addupdate_scatter(bset, [idx + 2 * _BINS], yv[sl])
            plsc.addupdate_scatter(bset, [idx + 3 * _BINS], zv[sl])
        return carry

    lax.fori_loop(0, _P // (16 * unroll), body, 0)
    for si, bset in enumerate((bins0, bins1)):
        for q in range(4):
            pltpu.sync_copy(
                bset.at[pl.ds(q * _BINS, _BINS)],
                out_ref.at[pl.ds(((q * 2 + si) * _NW + wid) * _BINS, _BINS)])


def _sc_stats(tgt_flat, pred_flat, b):
    mesh = plsc.VectorSubcoreMesh(core_axis_name="c", subcore_axis_name="s",
                                  num_cores=2, num_subcores=16)
    vm = pltpu.VMEM
    call = pl.kernel(
        functools.partial(_sc_stats_body, b),
        out_type=jax.ShapeDtypeStruct((4 * 2 * _NW * _BINS,), jnp.float32),
        mesh=mesh,
        compiler_params=pltpu.CompilerParams(needs_layout_passes=False),
        scratch_types=[
            vm((_P,), jnp.int32), vm((_P,), jnp.int32), vm((_P,), jnp.int32),
            vm((_P,), jnp.float32), vm((_P,), jnp.float32), vm((_P,), jnp.float32),
            vm((4 * _BINS,), jnp.float32), vm((4 * _BINS,), jnp.float32),
            pltpu.SemaphoreType.DMA,
        ],
    )
    return call(tgt_flat, pred_flat).reshape(4, 2 * _NW, _BINS)


def _loss_body(b, nobg_ref, pred_ref, tgt_ref, part_ref, out_ref,
               stats_ref, acc_ref, sid_ref, hub_ref):
    i = pl.program_id(0)
    f32 = jnp.float32

    @pl.when(i == 0)
    def _combine():
        sid_ref[...] = tgt_ref[0, 0] * 9 + tgt_ref[0, 1] * 3 + tgt_ref[0, 2]
        ri = lax.broadcasted_iota(jnp.int32, (_BINS, 32), 0)
        ci = lax.broadcasted_iota(jnp.int32, (_BINS, 32), 1)
        fold = (lax.shift_right_logical(ri, 4) == ci).astype(f32)
        lanes32 = lax.broadcasted_iota(jnp.int32, (1, 32), 1)
        zv = jnp.zeros((1, 32), f32)
        for q in range(4):
            arr = part_ref[q]
            m1 = lax.dot_general(arr, fold, (((1,), (0,)), ((), ())),
                                 precision=lax.Precision.HIGHEST,
                                 preferred_element_type=f32)
            v = jnp.sum(m1, axis=0, keepdims=True)
            for j in range(_NI):
                stats_ref[j, q] = jnp.sum(jnp.where(lanes32 == j, v, zv))

    @pl.when(i == 1)
    def _gather_huber():
        mus = [(f32(0.0), f32(0.0), f32(0.0))]
        stats_ref[0, 4] = f32(0.0)
        stats_ref[0, 5] = f32(0.0)
        stats_ref[0, 6] = f32(0.0)
        for j in range(1, _NI):
            safe = jnp.maximum(stats_ref[j, 0], 1.0)
            mj = (stats_ref[j, 1] / safe,
                  stats_ref[j, 2] / safe,
                  stats_ref[j, 3] / safe)
            stats_ref[j, 4] = mj[0]
            stats_ref[j, 5] = mj[1]
            stats_ref[j, 6] = mj[2]
            mus.append(mj)
        for c in range(_NCH):
            sl = pl.ds(c * _CHUNK, _CHUNK)
            sid = sid_ref[sl]
            zc = jnp.zeros((_CHUNK, _LANES), f32)
            mx, my, mz = zc, zc, zc
            for j in range(1, _NI):
                m = sid == j
                mx = jnp.where(m, mus[j][0], mx)
                my = jnp.where(m, mus[j][1], my)
                mz = jnp.where(m, mus[j][2], mz)
            dx = pred_ref[0, 0, sl] - mx
            dy = pred_ref[0, 1, sl] - my
            dz = pred_ref[0, 2, sl] - mz
            adx = jnp.abs(dx)
            ady = jnp.abs(dy)
            adz = jnp.abs(dz)
            nx = jnp.minimum(adx, 1.0)
            ny = jnp.minimum(ady, 1.0)
            nz = jnp.minimum(adz, 1.0)
            hub = (nx * (2.0 * adx - nx) + ny * (2.0 * ady - ny)
                   + nz * (2.0 * adz - nz))
            hub_ref[sl] = 0.5 * hub

    @pl.when(i > 1)
    def _dense():
        j = i - 2
        cnt = stats_ref[j, 0]
        mex = stats_ref[j, 4]
        mey = stats_ref[j, 5]
        mez = stats_ref[j, 6]
        zc = jnp.zeros((_CHUNK, _LANES), f32)
        sa, ha, oa = zc, zc, zc
        for c in range(_NCH):
            sl = pl.ds(c * _CHUNK, _CHUNK)
            m = sid_ref[sl] == j
            dx = pred_ref[0, 0, sl] - mex
            dy = pred_ref[0, 1, sl] - mey
            dz = pred_ref[0, 2, sl] - mez
            dist = dx * dx + dy * dy + dz * dz
            fr = 300.0 / (1.0 + dist)
            sa = sa + fr
            ha = ha + jnp.where(m, hub_ref[sl], zc)
            oa = oa + jnp.where(m, fr, zc)
        Sj = jnp.sum(sa)
        Hj = jnp.sum(ha)
        OWNj = jnp.sum(oa)
        lanes = lax.broadcasted_iota(jnp.int32, (1, _LANES), 1)
        lm = lanes == j
        acc_ref[0:1] = jnp.where(lm, cnt, acc_ref[0:1])
        acc_ref[1:2] = jnp.where(lm, Hj, acc_ref[1:2])
        acc_ref[2:3] = jnp.where(lm, Sj, acc_ref[2:3])
        acc_ref[3:4] = jnp.where(lm, OWNj, acc_ref[3:4])
        acc_ref[4:5] = jnp.where(lm, mex, acc_ref[4:5])
        acc_ref[5:6] = jnp.where(lm, mey, acc_ref[5:6])
        acc_ref[6:7] = jnp.where(lm, mez, acc_ref[6:7])

        @pl.when(j == _NI - 1)
        def _assemble():
            lanes1 = lax.broadcasted_iota(jnp.int32, (1, _LANES), 1)
            inrange = lanes1 < _NI
            nobg_ok = nobg_ref[b] == 0
            cntv = acc_ref[0:1]
            Hv = acc_ref[1:2]
            Sv = acc_ref[2:3]
            OWNv = acc_ref[3:4]
            safev = jnp.maximum(cntv, 1.0)
            presentv = jnp.logical_and(cntv > 0.0, inrange)
            hmask = jnp.logical_and(presentv,
                                    jnp.logical_or(lanes1 > 0, nobg_ok))
            hterm = Hv / (safev * 3.0)
            ncv = f32(_N) - cntv
            sepv = ((Sv - OWNv) / jnp.maximum(ncv, 1.0)) * (10.0 / jnp.sqrt(safev))
            sepmask = jnp.logical_and(
                jnp.logical_and(presentv, ncv > 0.0), lanes1 > 0)
            zl = jnp.zeros_like(hterm)
            vv = jnp.where(hmask, jnp.ones_like(hterm), zl)
            loss = jnp.sum(jnp.where(hmask, hterm, zl)
                           + jnp.where(sepmask, sepv, zl))
            ct = jnp.sum(vv)

            onesv = jnp.ones((1, _LANES), f32)

            def _outer(v):
                return lax.dot_general(v, onesv, (((0,), (0,)), ((), ())),
                                       precision=lax.Precision.HIGHEST,
                                       preferred_element_type=f32)

            mxv = acc_ref[4:5]
            myv = acc_ref[5:6]
            mzv = acc_ref[6:7]
            ddx = _outer(mxv) - jnp.broadcast_to(mxv, (_LANES, _LANES))
            ddy = _outer(myv) - jnp.broadcast_to(myv, (_LANES, _LANES))
            ddz = _outer(mzv) - jnp.broadcast_to(mzv, (_LANES, _LANES))
            sq = ddx * ddx + ddy * ddy + ddz * ddz
            vcol = _outer(vv)
            vrow = jnp.broadcast_to(vv, (_LANES, _LANES))
            pv = vcol * vrow
            ri = lax.broadcasted_iota(jnp.int32, (_LANES, _LANES), 0)
            ci = lax.broadcasted_iota(jnp.int32, (_LANES, _LANES), 1)
            upper = jnp.logical_and(ri < ci, ci < _NI)
            zz = jnp.zeros_like(sq)
            pair_sum = jnp.sum(jnp.where(upper, (300.0 / (sq + 1.0)) * pv, zz))
            npair = jnp.sum(jnp.where(upper, pv, zz))
            pair_term = pair_sum / jnp.maximum(npair, 1.0)

            lossb = loss + jnp.where(ct > 1.0, pair_term, 0.0)
            out_ref[...] = jnp.full((8, _LANES),
                                    lossb / jnp.maximum(ct, 1.0), f32)


def _make_call(b, interpret=False):
    return pl.pallas_call(
        functools.partial(_loss_body, b),
        grid=(_NI + 2,),
        out_shape=jax.ShapeDtypeStruct((8, _LANES), jnp.float32),
        in_specs=[
            pl.BlockSpec(memory_space=pltpu.SMEM),
            pl.BlockSpec((1, 3, _ROWS, _LANES), lambda i, _b=b: (_b, 0, 0, 0)),
            pl.BlockSpec((1, 3, _ROWS, _LANES), lambda i, _b=b: (_b, 0, 0, 0)),
            pl.BlockSpec((4, 2 * _NW, _BINS), lambda i: (0, 0, 0)),
        ],
        out_specs=pl.BlockSpec((8, _LANES), lambda i: (0, 0)),
        scratch_shapes=[
            pltpu.SMEM((32, 8), jnp.float32),
            pltpu.VMEM((8, _LANES), jnp.float32),
            pltpu.VMEM((_ROWS, _LANES), jnp.int32),
            pltpu.VMEM((_ROWS, _LANES), jnp.float32),
        ],
        interpret=interpret,
    )


def kernel(prediction, target, no_bg):
    pred = prediction.astype(jnp.float32).reshape(2, 3, _ROWS, _LANES)
    tgt = target.astype(jnp.int32).reshape(2, 3, _ROWS, _LANES)
    tgt_flat = target.astype(jnp.int32).reshape(-1)
    pred_flat = prediction.astype(jnp.float32).reshape(-1)
    nobg = no_bg.astype(jnp.int32)
    part0 = _sc_stats(tgt_flat, pred_flat, 0)
    part1 = _sc_stats(tgt_flat, pred_flat, 1)
    o0 = _make_call(0)(nobg, pred, tgt, part0)
    o1 = _make_call(1)(nobg, pred, tgt, part1)
    return (o0[0, 0] + o1[0, 0]) * 0.5
